# baseline (speedup 1.0000x reference)
import functools

import jax
import jax.numpy as jnp
import numpy as np
from jax import lax
from jax.experimental import pallas as pl
from jax.experimental.pallas import tpu as pltpu
from jax.experimental.pallas import tpu_sc as plsc

_N = 1024
_ROWS = 16384
_NW = 32
_RPW = _ROWS // _NW
_R = 16
_NCHUNK = _RPW // _R
_L = 16
_G = 16

_PERM = np.arange(_N)
np.random.RandomState(0).shuffle(_PERM)
_PERM = _PERM.astype(np.int32)


def _sc_permute(in_hbm, perm_hbm, out_hbm,
                perm_v, in0, in1, out0, out1, isem, osem):
    wid = lax.axis_index("s") * 2 + lax.axis_index("c")
    row0 = wid * _RPW
    pltpu.sync_copy(perm_hbm, perm_v)
    inbufs = (in0, in1)
    outbufs = (out0, out1)

    def in_desc(c, b):
        return pltpu.make_async_copy(
            in_hbm.at[pl.ds(row0 + c * _R, _R)], inbufs[b], isem)

    def out_desc(d, ob):
        return pltpu.make_async_copy(
            outbufs[ob], out_hbm.at[pl.ds(row0 + d * 2 * _R, 2 * _R)], osem)

    def compute(inbuf, outbuf, half):
        def group_body(g, carry):
            col0 = g * (_L * _G)
            idxs = [perm_v[pl.ds(col0 + _L * t, _L)] for t in range(_G)]

            def row_body(r):
                rv = jnp.zeros((_L,), jnp.int32) + r
                for t in range(_G):
                    v = plsc.load_gather(inbuf, [rv, idxs[t]])
                    outbuf[half * _R + r, pl.ds(col0 + _L * t, _L)] = v

            plsc.parallel_loop(0, _R, 1, unroll=4)(row_body)
            return carry

        lax.fori_loop(0, _N // (_L * _G), group_body, 0)

    def run_chunk(c, b4, *, wait_prev_out, start_next_in):
        b = b4 % 2
        half = b4 % 2
        ob = b4 // 2
        in_desc(c, b).wait()
        if half == 0 and wait_prev_out:
            out_desc(c // 2 - 2, ob).wait()
        compute(inbufs[b], outbufs[ob], half)
        if half == 1:
            out_desc(c // 2, ob).start()
        if start_next_in:
            in_desc(c + 2, b).start()

    in_desc(0, 0).start()
    in_desc(1, 1).start()
    for b in range(4):
        run_chunk(b, b, wait_prev_out=False, start_next_in=True)

    def quad_body(q, carry):
        c = 4 * q
        for b in range(4):
            run_chunk(c + b, b, wait_prev_out=True, start_next_in=True)
        return carry

    lax.fori_loop(1, _NCHUNK // 4 - 1, quad_body, 0)

    for b in range(4):
        run_chunk(_NCHUNK - 4 + b, b, wait_prev_out=True, start_next_in=(b < 2))
    out_desc(_NCHUNK // 2 - 2, 0).wait()
    out_desc(_NCHUNK // 2 - 1, 1).wait()


@jax.jit
def _run(inputs):
    mesh = plsc.VectorSubcoreMesh(core_axis_name="c", subcore_axis_name="s")
    fn = functools.partial(
        pl.kernel,
        mesh=mesh,
        out_type=jax.ShapeDtypeStruct((_ROWS, _N), jnp.float32),
        scratch_types=[
            pltpu.VMEM((_N,), jnp.int32),
            pltpu.VMEM((_R, _N), jnp.float32),
            pltpu.VMEM((_R, _N), jnp.float32),
            pltpu.VMEM((2 * _R, _N), jnp.float32),
            pltpu.VMEM((2 * _R, _N), jnp.float32),
            pltpu.SemaphoreType.DMA,
            pltpu.SemaphoreType.DMA,
        ],
        compiler_params=pltpu.CompilerParams(needs_layout_passes=False),
    )(_sc_permute)
    return fn(inputs, jnp.asarray(_PERM))


def kernel(inputs):
    out = _run(inputs)
    logdet = jnp.zeros((_ROWS, 1), dtype=inputs.dtype)
    return (out, logdet)


# speedup vs baseline: 1.0805x; 1.0805x over previous
import functools

import jax
import jax.numpy as jnp
import numpy as np
from jax import lax
from jax.experimental import pallas as pl
from jax.experimental.pallas import tpu as pltpu
from jax.experimental.pallas import tpu_sc as plsc

_N = 1024
_ROWS = 16384
_NW = 32
_RPW = _ROWS // _NW
_R = 16
_NCHUNK = _RPW // _R
_L = 16
_G = 8

_PERM = np.arange(_N)
np.random.RandomState(0).shuffle(_PERM)
_PERM = _PERM.astype(np.int32)


def _sc_permute(in_hbm, perm_hbm, out_hbm,
                perm_v, in0, in1, out0, out1, isem, osem):
    wid = lax.axis_index("s") * 2 + lax.axis_index("c")
    row0 = wid * _RPW
    pltpu.sync_copy(perm_hbm, perm_v)
    inbufs = (in0, in1)
    outbufs = (out0, out1)

    def in_desc(c, b):
        return pltpu.make_async_copy(
            in_hbm.at[pl.ds(row0 + c * _R, _R)], inbufs[b], isem)

    def out_desc(d, ob):
        return pltpu.make_async_copy(
            outbufs[ob], out_hbm.at[pl.ds(row0 + d * 2 * _R, 2 * _R)], osem)

    def compute(inbuf, outbuf, half):
        def group_body(g, carry):
            col0 = g * (_L * _G)
            idxs = [perm_v[pl.ds(col0 + _L * t, _L)] for t in range(_G)]

            def row_body(r):
                rv = jnp.zeros((_L,), jnp.int32) + r
                for t in range(_G):
                    v = plsc.load_gather(inbuf, [rv, idxs[t]])
                    outbuf[half * _R + r, pl.ds(col0 + _L * t, _L)] = v

            plsc.parallel_loop(0, _R, 1, unroll=2)(row_body)
            return carry

        lax.fori_loop(0, _N // (_L * _G), group_body, 0)

    def run_chunk(c, b4, *, wait_prev_out, start_next_in):
        b = b4 % 2
        half = b4 % 2
        ob = b4 // 2
        in_desc(c, b).wait()
        if half == 0 and wait_prev_out:
            out_desc(c // 2 - 2, ob).wait()
        compute(inbufs[b], outbufs[ob], half)
        if half == 1:
            out_desc(c // 2, ob).start()
        if start_next_in:
            in_desc(c + 2, b).start()

    in_desc(0, 0).start()
    in_desc(1, 1).start()
    for b in range(4):
        run_chunk(b, b, wait_prev_out=False, start_next_in=True)

    def quad_body(q, carry):
        c = 4 * q
        for b in range(4):
            run_chunk(c + b, b, wait_prev_out=True, start_next_in=True)
        return carry

    lax.fori_loop(1, _NCHUNK // 4 - 1, quad_body, 0)

    for b in range(4):
        run_chunk(_NCHUNK - 4 + b, b, wait_prev_out=True, start_next_in=(b < 2))
    out_desc(_NCHUNK // 2 - 2, 0).wait()
    out_desc(_NCHUNK // 2 - 1, 1).wait()


@jax.jit
def _run(inputs):
    mesh = plsc.VectorSubcoreMesh(core_axis_name="c", subcore_axis_name="s")
    fn = functools.partial(
        pl.kernel,
        mesh=mesh,
        out_type=jax.ShapeDtypeStruct((_ROWS, _N), jnp.float32),
        scratch_types=[
            pltpu.VMEM((_N,), jnp.int32),
            pltpu.VMEM((_R, _N), jnp.float32),
            pltpu.VMEM((_R, _N), jnp.float32),
            pltpu.VMEM((2 * _R, _N), jnp.float32),
            pltpu.VMEM((2 * _R, _N), jnp.float32),
            pltpu.SemaphoreType.DMA,
            pltpu.SemaphoreType.DMA,
        ],
        compiler_params=pltpu.CompilerParams(needs_layout_passes=False),
    )(_sc_permute)
    return fn(inputs, jnp.asarray(_PERM))


def kernel(inputs):
    out = _run(inputs)
    logdet = jnp.zeros((_ROWS, 1), dtype=inputs.dtype)
    return (out, logdet)
